# baseline (speedup 1.0000x reference)
import functools

import jax
import jax.numpy as jnp
from jax import lax
from jax.experimental import pallas as pl
from jax.experimental.pallas import tpu as pltpu
from jax.experimental.pallas import tpu_sc as plsc

_N = 16
_MODES = 2
_OUT_RES = 32
_N0 = 1.0
_EY = 192
_TOTAL = 672
_NW = 32
_PAGES = _N * _N * _MODES
_PPW = _PAGES // _NW


def _sc_body(ey_hbm, out_hbm, buf0, buf1, sem0, sem1):
    wid = lax.axis_index("c") * 16 + lax.axis_index("s")
    base = wid * _PPW
    bufs = (buf0, buf1)
    sems = (sem0, sem1)

    def fetch(p, slot):
        c = p // _MODES
        m = p % _MODES
        return pltpu.async_copy(ey_hbm.at[c, m], bufs[slot], sems[slot])

    def drain(p, slot):
        c = p // _MODES
        m = p % _MODES
        return pltpu.async_copy(bufs[slot], out_hbm.at[c, m], sems[slot])

    fetch(base, 0).wait()
    for k in range(_PPW):
        slot = k % 2
        d = drain(base + k, slot)
        if k + 1 < _PPW:
            fetch(base + k + 1, 1 - slot).wait()
        d.wait()


@functools.partial(
    pl.kernel,
    mesh=plsc.VectorSubcoreMesh(core_axis_name="c", subcore_axis_name="s"),
    out_type=jax.ShapeDtypeStruct((_N * _N, _MODES, _EY, _EY), jnp.float32),
    scratch_types=[
        pltpu.VMEM((_EY, _EY), jnp.float32),
        pltpu.VMEM((_EY, _EY), jnp.float32),
        pltpu.SemaphoreType.DMA,
        pltpu.SemaphoreType.DMA,
    ],
)
def _sc_reformat(ey_hbm, out_hbm, buf0, buf1, sem0, sem1):
    _sc_body(ey_hbm, out_hbm, buf0, buf1, sem0, sem1)


def _tc_body(u_ref, neff_ref, ey_ref, out_ref, acc_ref):
    i = pl.program_id(0)

    @pl.when(i == 0)
    def _():
        acc_ref[...] = jnp.zeros_like(acc_ref)

    r0 = i * _OUT_RES
    for j in range(_N):
        c = i * _N + j
        n0_ = neff_ref[c, 0]
        n1_ = neff_ref[c, 1]
        w0 = (n0_ * _N0 / (n0_ + _N0)) * u_ref[c, 0]
        w1 = (n1_ * _N0 / (n1_ + _N0)) * u_ref[c, 1]
        patch = ey_ref[j, 0] * w0 + ey_ref[j, 1] * w1
        acc_ref[pl.ds(r0, _EY), j * _OUT_RES:j * _OUT_RES + _EY] += patch

    @pl.when(i == pl.num_programs(0) - 1)
    def _():
        out_ref[...] = acc_ref[...]


def kernel(hs, U, neff, Ey):
    del hs
    ey_c = _sc_reformat(Ey)
    en = pl.pallas_call(
        _tc_body,
        grid=(_N,),
        in_specs=[
            pl.BlockSpec(memory_space=pltpu.SMEM),
            pl.BlockSpec(memory_space=pltpu.SMEM),
            pl.BlockSpec((_N, _MODES, _EY, _EY),
                         lambda i: (i, 0, 0, 0)),
        ],
        out_specs=pl.BlockSpec((_TOTAL, _TOTAL), lambda i: (0, 0)),
        out_shape=jax.ShapeDtypeStruct((_TOTAL, _TOTAL), jnp.float32),
        scratch_shapes=[pltpu.VMEM((_TOTAL, _TOTAL), jnp.float32)],
    )(U, neff, ey_c)
    return en.astype(jnp.complex64)


# speedup vs baseline: 1.2663x; 1.2663x over previous
import jax
import jax.numpy as jnp
from jax.experimental import pallas as pl
from jax.experimental.pallas import tpu as pltpu

_N = 16
_MODES = 2
_OUT_RES = 32
_N0 = 1.0
_EY = 192
_TOTAL = 672
_SPLIT = 8


def _make_body(i0, have_acc_in):
    def _body(u_ref, neff_ref, ey_ref, *rest):
        if have_acc_in:
            acc_in_ref, out_ref, acc_ref = rest
        else:
            out_ref, acc_ref = rest
        g = pl.program_id(0)

        @pl.when(g == 0)
        def _():
            if have_acc_in:
                acc_ref[...] = acc_in_ref[...]
            else:
                acc_ref[...] = jnp.zeros_like(acc_ref)

        i = i0 + g
        r0 = i * _OUT_RES
        for j in range(_N):
            c = i * _N + j
            n0_ = neff_ref[c, 0]
            n1_ = neff_ref[c, 1]
            w0 = (n0_ * _N0 / (n0_ + _N0)) * u_ref[c, 0]
            w1 = (n1_ * _N0 / (n1_ + _N0)) * u_ref[c, 1]
            patch = ey_ref[0, j, 0] * w0 + ey_ref[0, j, 1] * w1
            acc_ref[pl.ds(r0, _EY), j * _OUT_RES:j * _OUT_RES + _EY] += patch

        @pl.when(g == pl.num_programs(0) - 1)
        def _():
            out_ref[...] = acc_ref[...]

    return _body


def kernel(hs, U, neff, Ey):
    del hs
    ey_fmt = Ey.reshape(_N, _N, _MODES, _EY, _EY)
    ey_raw = Ey[:_SPLIT * _N].reshape(_SPLIT, _N, _MODES, _EY, _EY)

    acc = pl.pallas_call(
        _make_body(0, False),
        grid=(_SPLIT,),
        in_specs=[
            pl.BlockSpec(memory_space=pltpu.SMEM),
            pl.BlockSpec(memory_space=pltpu.SMEM),
            pl.BlockSpec((1, _N, _MODES, _EY, _EY),
                         lambda g: (g, 0, 0, 0, 0)),
        ],
        out_specs=pl.BlockSpec((_TOTAL, _TOTAL), lambda g: (0, 0)),
        out_shape=jax.ShapeDtypeStruct((_TOTAL, _TOTAL), jnp.float32),
        scratch_shapes=[pltpu.VMEM((_TOTAL, _TOTAL), jnp.float32)],
    )(U, neff, ey_raw)

    en = pl.pallas_call(
        _make_body(_SPLIT, True),
        grid=(_N - _SPLIT,),
        in_specs=[
            pl.BlockSpec(memory_space=pltpu.SMEM),
            pl.BlockSpec(memory_space=pltpu.SMEM),
            pl.BlockSpec((1, _N, _MODES, _EY, _EY),
                         lambda g: (g + _SPLIT, 0, 0, 0, 0)),
            pl.BlockSpec((_TOTAL, _TOTAL), lambda g: (0, 0)),
        ],
        out_specs=pl.BlockSpec((_TOTAL, _TOTAL), lambda g: (0, 0)),
        out_shape=jax.ShapeDtypeStruct((_TOTAL, _TOTAL), jnp.float32),
        scratch_shapes=[pltpu.VMEM((_TOTAL, _TOTAL), jnp.float32)],
        input_output_aliases={3: 0},
    )(U, neff, ey_fmt, acc)
    return en.astype(jnp.complex64)


# speedup vs baseline: 1.8435x; 1.4558x over previous
import jax
import jax.numpy as jnp
from jax.experimental import pallas as pl
from jax.experimental.pallas import tpu as pltpu

_N = 16
_MODES = 2
_OUT_RES = 32
_N0 = 1.0
_EY = 192
_TOTAL = 672


def _body(u_ref, neff_ref, ey0_ref, ey1_ref, out_ref, acc_ref):
    i = pl.program_id(0)

    @pl.when(i == 0)
    def _():
        acc_ref[...] = jnp.zeros_like(acc_ref)

    r0 = i * _OUT_RES
    for j in range(_N):
        ey_ref = ey0_ref if j < _N // 2 else ey1_ref
        jj = j % (_N // 2)
        c = i * _N + j
        n0_ = neff_ref[c, 0]
        n1_ = neff_ref[c, 1]
        w0 = (n0_ * _N0 / (n0_ + _N0)) * u_ref[c, 0]
        w1 = (n1_ * _N0 / (n1_ + _N0)) * u_ref[c, 1]
        patch = ey_ref[0, jj, 0] * w0 + ey_ref[0, jj, 1] * w1
        acc_ref[pl.ds(r0, _EY), j * _OUT_RES:j * _OUT_RES + _EY] += patch

    @pl.when(i == pl.num_programs(0) - 1)
    def _():
        out_ref[...] = acc_ref[...]


def kernel(hs, U, neff, Ey):
    del hs
    en = pl.pallas_call(
        _body,
        grid=(_N,),
        in_specs=[
            pl.BlockSpec(memory_space=pltpu.SMEM),
            pl.BlockSpec(memory_space=pltpu.SMEM),
            pl.BlockSpec((1, _N // 2, _MODES, _EY, _EY),
                         lambda i: (i, 0, 0, 0, 0)),
            pl.BlockSpec((1, _N // 2, _MODES, _EY, _EY),
                         lambda i: (i, 1, 0, 0, 0)),
        ],
        out_specs=pl.BlockSpec((_TOTAL, _TOTAL), lambda i: (0, 0)),
        out_shape=jax.ShapeDtypeStruct((_TOTAL, _TOTAL), jnp.float32),
        scratch_shapes=[pltpu.VMEM((_TOTAL, _TOTAL), jnp.float32)],
    )(U, neff, *([Ey.reshape(_N, _N, _MODES, _EY, _EY)] * 2))
    return en.astype(jnp.complex64)


# speedup vs baseline: 1.8458x; 1.0013x over previous
import jax
import jax.numpy as jnp
from jax.experimental import pallas as pl
from jax.experimental.pallas import tpu as pltpu

_N = 16
_MODES = 2
_OUT_RES = 32
_N0 = 1.0
_EY = 192
_TOTAL = 672


def _body(u_ref, neff_ref, ey_ref, out_ref, acc_ref):
    i = pl.program_id(0)

    @pl.when(i == 0)
    def _():
        acc_ref[...] = jnp.zeros_like(acc_ref)

    r0 = i * _OUT_RES
    for j in range(_N):
        c = i * _N + j
        n0_ = neff_ref[c, 0]
        n1_ = neff_ref[c, 1]
        w0 = (n0_ * _N0 / (n0_ + _N0)) * u_ref[c, 0]
        w1 = (n1_ * _N0 / (n1_ + _N0)) * u_ref[c, 1]
        patch = ey_ref[0, j, 0] * w0 + ey_ref[0, j, 1] * w1
        acc_ref[pl.ds(r0, _EY), j * _OUT_RES:j * _OUT_RES + _EY] += patch

    @pl.when(i == pl.num_programs(0) - 1)
    def _():
        out_ref[...] = acc_ref[...]


def kernel(hs, U, neff, Ey):
    del hs
    en = pl.pallas_call(
        _body,
        grid=(_N,),
        in_specs=[
            pl.BlockSpec(memory_space=pltpu.SMEM),
            pl.BlockSpec(memory_space=pltpu.SMEM),
            pl.BlockSpec((1, _N, _MODES, _EY, _EY),
                         lambda i: (i, 0, 0, 0, 0)),
        ],
        out_specs=pl.BlockSpec((_TOTAL, _TOTAL), lambda i: (0, 0)),
        out_shape=jax.ShapeDtypeStruct((_TOTAL, _TOTAL), jnp.float32),
        scratch_shapes=[pltpu.VMEM((_TOTAL, _TOTAL), jnp.float32)],
    )(U, neff, Ey.reshape(_N, _N, _MODES, _EY, _EY))
    return en.astype(jnp.complex64)
